# baseline (speedup 1.0000x reference)
import functools

import jax
import jax.numpy as jnp
from jax import lax
from jax.experimental import pallas as pl
from jax.experimental.pallas import tpu as pltpu
from jax.experimental.pallas import tpu_sc as plsc

N = 10000
D = 128
DAUG = 144
NT = 10008
E = 320000
NC = 2
NS = 16
CHUNK = 128
G = 16
NG = 5
CHUNKS = NG * G
EPAD = NC * NS * CHUNKS * CHUNK
NACC = N
ROWS_LO = 624
GROW = 256

BLK_A = 1112
BLK_C = 1000


def _proj_body(x_ref, wpt_ref, bp_ref, out_ref):
    r = pl.program_id(0)
    hv = jnp.dot(x_ref[...], wpt_ref[...], preferred_element_type=jnp.float32)
    hv = jnp.maximum(hv + bp_ref[...], 0.0)
    rows = r * BLK_A + lax.broadcasted_iota(jnp.int32, (BLK_A, 1), 0)
    hb = jnp.concatenate(
        [hv, jnp.ones((BLK_A, DAUG - D), jnp.float32)], axis=1)
    out_ref[...] = jnp.where(rows < N, hb, 0.0)


_proj = pl.pallas_call(
    _proj_body,
    grid=(NT // BLK_A,),
    in_specs=[
        pl.BlockSpec((BLK_A, D), lambda r: (r, 0)),
        pl.BlockSpec((D, D), lambda r: (0, 0)),
        pl.BlockSpec((1, D), lambda r: (0, 0)),
    ],
    out_specs=pl.BlockSpec((BLK_A, DAUG), lambda r: (r, 0)),
    out_shape=jax.ShapeDtypeStruct((NT, DAUG), jnp.float32),
)


def _sc_agg(src_hbm, dst_hbm, haug_hbm, out_hbm, src_a, dst_a,
            rows0, acc, sem0):
    c = lax.axis_index("c")
    s = lax.axis_index("s")

    def zero_body(i, carry):
        for j in range(DAUG // 16):
            rows0[i, pl.ds(j * 16, 16)] = jnp.zeros((16,), jnp.float32)
        return carry

    lax.fori_loop(0, GROW, zero_body, 0)
    for z in range(2):
        pltpu.sync_copy(rows0, acc.at[pl.ds(s * ROWS_LO + z * GROW, GROW)])

    @pl.when(s < NS - 1)
    def _():
        pltpu.sync_copy(rows0.at[pl.ds(0, ROWS_LO - 512)],
                        acc.at[pl.ds(s * ROWS_LO + 512, ROWS_LO - 512)])

    @pl.when(s == NS - 1)
    def _():
        pltpu.sync_copy(rows0.at[pl.ds(0, CHUNK)],
                        acc.at[pl.ds((NS - 1) * ROWS_LO + 512, CHUNK)])

    plsc.subcore_barrier()

    for sg in range(NG):
        pltpu.sync_copy(src_hbm.at[c, s, sg], src_a)
        pltpu.sync_copy(dst_hbm.at[c, s, sg], dst_a)

        def q_body(q, carry):
            pltpu.async_copy(haug_hbm.at[src_a.at[pl.ds(q * GROW, GROW)]],
                             rows0, sem0).wait()
            pltpu.sync_copy(rows0.at[pl.ds(0, CHUNK)],
                            acc.at[dst_a.at[2 * q]], add=True)
            pltpu.sync_copy(rows0.at[pl.ds(CHUNK, CHUNK)],
                            acc.at[dst_a.at[2 * q + 1]], add=True)
            return carry

        lax.fori_loop(0, (G * CHUNK) // GROW, q_body, 0)
    plsc.subcore_barrier()

    for z in range(2):
        base = s * ROWS_LO + z * GROW
        pltpu.sync_copy(acc.at[pl.ds(base, GROW)], rows0)
        pltpu.sync_copy(rows0, out_hbm.at[c, pl.ds(base, GROW)])

    @pl.when(s < NS - 1)
    def _():
        base = s * ROWS_LO + 512
        pltpu.sync_copy(acc.at[pl.ds(base, ROWS_LO - 512)],
                        rows0.at[pl.ds(0, ROWS_LO - 512)])
        pltpu.sync_copy(rows0.at[pl.ds(0, ROWS_LO - 512)],
                        out_hbm.at[c, pl.ds(base, ROWS_LO - 512)])

    @pl.when(s == NS - 1)
    def _():
        base = (NS - 1) * ROWS_LO + 512
        pltpu.sync_copy(acc.at[pl.ds(base, CHUNK)], rows0.at[pl.ds(0, CHUNK)])
        pltpu.sync_copy(rows0.at[pl.ds(0, CHUNK)],
                        out_hbm.at[c, pl.ds(base, CHUNK)])


@functools.lru_cache(maxsize=1)
def _sc_agg_call():
    mesh = plsc.VectorSubcoreMesh(core_axis_name="c", subcore_axis_name="s",
                                  num_cores=NC, num_subcores=NS)
    return pl.kernel(
        _sc_agg,
        out_type=jax.ShapeDtypeStruct((NC, NACC, DAUG), jnp.float32),
        mesh=mesh,
        scratch_types=[
            pltpu.VMEM((G * CHUNK,), jnp.int32),
            pltpu.VMEM((G, CHUNK), jnp.int32),
            pltpu.VMEM((GROW, DAUG), jnp.float32),
            pltpu.VMEM_SHARED((NACC, DAUG), jnp.float32),
            pltpu.SemaphoreType.DMA,
        ],
        compiler_params=pltpu.CompilerParams(use_tc_tiling_on_sc=False),
    )


def _post_body(acc_ref, haug_ref, wlt_ref, bl_ref, wrt_ref, g_ref, be_ref,
               wot_ref, bo_ref, y_ref, gsum):
    r = pl.program_id(0)
    a0 = acc_ref[0]
    a1 = acc_ref[1]
    summed = a0[:, :D] + a1[:, :D]
    cnt = a0[:, D:D + 1] + a1[:, D:D + 1]
    agg = summed * (1.0 / jnp.maximum(cnt, 1.0))
    h = haug_ref[:, :D]
    out = (jnp.dot(agg, wlt_ref[...], preferred_element_type=jnp.float32)
           + bl_ref[...]
           + jnp.dot(h, wrt_ref[...], preferred_element_type=jnp.float32))
    mu = jnp.mean(out, axis=1, keepdims=True)
    dlt = out - mu
    var = jnp.mean(dlt * dlt, axis=1, keepdims=True)
    hn = dlt * lax.rsqrt(var + 1e-5) * g_ref[...] + be_ref[...]
    hn = jnp.maximum(hn, 0.0)
    part = jnp.sum(hn, axis=0, keepdims=True)

    @pl.when(r == 0)
    def _():
        gsum[...] = part

    @pl.when(r > 0)
    def _():
        gsum[...] = gsum[...] + part

    @pl.when(r == pl.num_programs(0) - 1)
    def _():
        g = gsum[...] * (1.0 / N)
        y_ref[...] = (jnp.dot(g, wot_ref[...],
                              preferred_element_type=jnp.float32)
                      + bo_ref[...])


_post = pl.pallas_call(
    _post_body,
    grid=(N // BLK_C,),
    in_specs=[
        pl.BlockSpec((NC, BLK_C, DAUG), lambda r: (0, r, 0)),
        pl.BlockSpec((BLK_C, DAUG), lambda r: (r, 0)),
        pl.BlockSpec((D, D), lambda r: (0, 0)),
        pl.BlockSpec((1, D), lambda r: (0, 0)),
        pl.BlockSpec((D, D), lambda r: (0, 0)),
        pl.BlockSpec((1, D), lambda r: (0, 0)),
        pl.BlockSpec((1, D), lambda r: (0, 0)),
        pl.BlockSpec((D, D), lambda r: (0, 0)),
        pl.BlockSpec((1, D), lambda r: (0, 0)),
    ],
    out_specs=pl.BlockSpec((1, D), lambda r: (0, 0)),
    out_shape=jax.ShapeDtypeStruct((1, D), jnp.float32),
    scratch_shapes=[pltpu.VMEM((1, D), jnp.float32)],
)


def kernel(x, edge_index, W_proj, b_proj, W_l, b_l, W_r, gamma, beta, W_out,
           b_out):
    haug = _proj(x, W_proj.T, b_proj.reshape(1, D))
    pad = EPAD - E
    srcp = jnp.concatenate(
        [edge_index[0], jnp.full((pad,), N, jnp.int32)]
    ).reshape(NC, NS, NG, G * CHUNK)
    dstp = jnp.concatenate(
        [edge_index[1], jnp.zeros((pad,), jnp.int32)]
    ).reshape(NC, NS, NG, G, CHUNK)
    acc = _sc_agg_call()(srcp, dstp, haug)
    return _post(acc, haug, W_l.T, b_l.reshape(1, D), W_r.T,
                 gamma.reshape(1, D), beta.reshape(1, D), W_out.T,
                 b_out.reshape(1, D))


# speedup vs baseline: 1.4330x; 1.4330x over previous
import functools

import jax
import jax.numpy as jnp
from jax import lax
from jax.experimental import pallas as pl
from jax.experimental.pallas import tpu as pltpu
from jax.experimental.pallas import tpu_sc as plsc

N = 10000
D = 128
DAUG = 144
NT = 10008
E = 320000
NC = 2
NS = 16
CHUNK = 128
CHUNKS = 79
EPAD = NC * NS * CHUNKS * CHUNK
NACC = 10240
ROWS_SUB = NACC // NS
WB = 128

BLK_A = 1112
BLK_C = 1000


def _proj_body(x_ref, wpt_ref, bp_ref, out_ref):
    r = pl.program_id(0)
    hv = jnp.dot(x_ref[...], wpt_ref[...], preferred_element_type=jnp.float32)
    hv = jnp.maximum(hv + bp_ref[...], 0.0)
    rows = r * BLK_A + lax.broadcasted_iota(jnp.int32, (BLK_A, 1), 0)
    hb = jnp.concatenate(
        [hv, jnp.ones((BLK_A, DAUG - D), jnp.float32)], axis=1)
    out_ref[...] = jnp.where(rows < N, hb, 0.0)


_proj = pl.pallas_call(
    _proj_body,
    grid=(NT // BLK_A,),
    in_specs=[
        pl.BlockSpec((BLK_A, D), lambda r: (r, 0)),
        pl.BlockSpec((D, D), lambda r: (0, 0)),
        pl.BlockSpec((1, D), lambda r: (0, 0)),
    ],
    out_specs=pl.BlockSpec((BLK_A, DAUG), lambda r: (r, 0)),
    out_shape=jax.ShapeDtypeStruct((NT, DAUG), jnp.float32),
)


def _sc_agg(src_hbm, dst_hbm, haug_hbm, out_hbm, src_v, dst_v, rows_v, acc,
            sem):
    c = lax.axis_index("c")
    s = lax.axis_index("s")

    def zero_body(i, carry):
        for j in range(DAUG // 16):
            rows_v[i, pl.ds(j * 16, 16)] = jnp.zeros((16,), jnp.float32)
        return carry

    lax.fori_loop(0, CHUNK, zero_body, 0)
    for z in range(ROWS_SUB // WB):
        base = s * ROWS_SUB + z * WB
        pltpu.sync_copy(rows_v, acc.at[pl.ds(base, WB)])
    plsc.subcore_barrier()

    pltpu.sync_copy(src_hbm.at[c, s], src_v)
    pltpu.sync_copy(dst_hbm.at[c, s], dst_v)

    def edge_body(j, carry):
        pltpu.async_copy(haug_hbm.at[src_v.at[j]], rows_v, sem).wait()
        pltpu.sync_copy(rows_v, acc.at[dst_v.at[j]], add=True)
        return carry

    lax.fori_loop(0, CHUNKS, edge_body, 0)
    plsc.subcore_barrier()

    for z in range(ROWS_SUB // WB):
        base = s * ROWS_SUB + z * WB
        pltpu.sync_copy(acc.at[pl.ds(base, WB)], rows_v)
        pltpu.sync_copy(rows_v, out_hbm.at[c, pl.ds(base, WB)])


@functools.lru_cache(maxsize=1)
def _sc_agg_call():
    mesh = plsc.VectorSubcoreMesh(core_axis_name="c", subcore_axis_name="s",
                                  num_cores=NC, num_subcores=NS)
    return pl.kernel(
        _sc_agg,
        out_type=jax.ShapeDtypeStruct((NC, NACC, DAUG), jnp.float32),
        mesh=mesh,
        scratch_types=[
            pltpu.VMEM((CHUNKS, CHUNK), jnp.int32),
            pltpu.VMEM((CHUNKS, CHUNK), jnp.int32),
            pltpu.VMEM((CHUNK, DAUG), jnp.float32),
            pltpu.VMEM_SHARED((NACC, DAUG), jnp.float32),
            pltpu.SemaphoreType.DMA,
        ],
        compiler_params=pltpu.CompilerParams(use_tc_tiling_on_sc=False),
    )


def _post_body(acc_ref, haug_ref, wlt_ref, bl_ref, wrt_ref, g_ref, be_ref,
               wot_ref, bo_ref, y_ref, gsum):
    r = pl.program_id(0)
    a0 = acc_ref[0]
    a1 = acc_ref[1]
    summed = a0[:, :D] + a1[:, :D]
    cnt = a0[:, D:D + 1] + a1[:, D:D + 1]
    agg = summed * (1.0 / jnp.maximum(cnt, 1.0))
    h = haug_ref[:, :D]
    out = (jnp.dot(agg, wlt_ref[...], preferred_element_type=jnp.float32)
           + bl_ref[...]
           + jnp.dot(h, wrt_ref[...], preferred_element_type=jnp.float32))
    mu = jnp.mean(out, axis=1, keepdims=True)
    dlt = out - mu
    var = jnp.mean(dlt * dlt, axis=1, keepdims=True)
    hn = dlt * lax.rsqrt(var + 1e-5) * g_ref[...] + be_ref[...]
    hn = jnp.maximum(hn, 0.0)
    part = jnp.sum(hn, axis=0, keepdims=True)

    @pl.when(r == 0)
    def _():
        gsum[...] = part

    @pl.when(r > 0)
    def _():
        gsum[...] = gsum[...] + part

    @pl.when(r == pl.num_programs(0) - 1)
    def _():
        g = gsum[...] * (1.0 / N)
        y_ref[...] = (jnp.dot(g, wot_ref[...],
                              preferred_element_type=jnp.float32)
                      + bo_ref[...])


_post = pl.pallas_call(
    _post_body,
    grid=(N // BLK_C,),
    in_specs=[
        pl.BlockSpec((NC, BLK_C, DAUG), lambda r: (0, r, 0)),
        pl.BlockSpec((BLK_C, DAUG), lambda r: (r, 0)),
        pl.BlockSpec((D, D), lambda r: (0, 0)),
        pl.BlockSpec((1, D), lambda r: (0, 0)),
        pl.BlockSpec((D, D), lambda r: (0, 0)),
        pl.BlockSpec((1, D), lambda r: (0, 0)),
        pl.BlockSpec((1, D), lambda r: (0, 0)),
        pl.BlockSpec((D, D), lambda r: (0, 0)),
        pl.BlockSpec((1, D), lambda r: (0, 0)),
    ],
    out_specs=pl.BlockSpec((1, D), lambda r: (0, 0)),
    out_shape=jax.ShapeDtypeStruct((1, D), jnp.float32),
    scratch_shapes=[pltpu.VMEM((1, D), jnp.float32)],
)


def kernel(x, edge_index, W_proj, b_proj, W_l, b_l, W_r, gamma, beta, W_out,
           b_out):
    haug = _proj(x, W_proj.T, b_proj.reshape(1, D))
    pad = EPAD - E
    srcp = jnp.concatenate(
        [edge_index[0], jnp.full((pad,), N, jnp.int32)]
    ).reshape(NC, NS, CHUNKS, CHUNK)
    dstp = jnp.concatenate(
        [edge_index[1], jnp.zeros((pad,), jnp.int32)]
    ).reshape(NC, NS, CHUNKS, CHUNK)
    acc = _sc_agg_call()(srcp, dstp, haug)
    return _post(acc, haug, W_l.T, b_l.reshape(1, D), W_r.T,
                 gamma.reshape(1, D), beta.reshape(1, D), W_out.T,
                 b_out.reshape(1, D))
